# Initial kernel scaffold; baseline (speedup 1.0000x reference)
import functools

import jax
import jax.numpy as jnp
from jax import lax
from jax.experimental import pallas as pl
from jax.experimental.pallas import tpu as pltpu
from jax.experimental.pallas import tpu_sc as plsc

N_NODES = 10000
N_EDGES = 320000
NPAD = 10240
NC, NS = 2, 16
NW = NC * NS
EPW = N_EDGES // NW
EBATCH = 80
NBATCH = EPW // EBATCH
RPT = NPAD // NS

_MESH = plsc.VectorSubcoreMesh(core_axis_name="c", subcore_axis_name="s")


@functools.partial(
    pl.kernel,
    out_type=jax.ShapeDtypeStruct((NC, 2, NPAD, 8), jnp.float32),
    mesh=_MESH,
    scratch_types=[
        pltpu.VMEM((EBATCH,), jnp.int32),
        pltpu.VMEM((EBATCH,), jnp.int32),
        pltpu.VMEM((EBATCH, 8), jnp.float32),
        pltpu.VMEM_SHARED((NPAD, 8), jnp.float32),
        pltpu.VMEM_SHARED((NPAD, 8), jnp.float32),
    ],
)
def _degree_kernel(src_hbm, dst_hbm, ones_hbm, zero8_hbm, out_hbm,
                   srcb, dstb, ones_v, dego_sh, degi_sh):
    cid = lax.axis_index("c")
    sid = lax.axis_index("s")
    wid = sid * NC + cid
    pltpu.sync_copy(zero8_hbm, dego_sh.at[pl.ds(sid * RPT, RPT)])
    pltpu.sync_copy(zero8_hbm, degi_sh.at[pl.ds(sid * RPT, RPT)])
    pltpu.sync_copy(ones_hbm, ones_v)
    plsc.subcore_barrier()

    base0 = wid * EPW

    def body(j, carry):
        base = base0 + j * EBATCH
        pltpu.sync_copy(src_hbm.at[pl.ds(base, EBATCH)], srcb)
        pltpu.sync_copy(dst_hbm.at[pl.ds(base, EBATCH)], dstb)
        pltpu.sync_copy(ones_v, dego_sh.at[srcb], add=True)
        pltpu.sync_copy(ones_v, degi_sh.at[dstb], add=True)
        return carry

    lax.fori_loop(0, NBATCH, body, 0)
    plsc.subcore_barrier()
    pltpu.sync_copy(dego_sh.at[pl.ds(sid * RPT, RPT)],
                    out_hbm.at[cid, 0, pl.ds(sid * RPT, RPT)])
    pltpu.sync_copy(degi_sh.at[pl.ds(sid * RPT, RPT)],
                    out_hbm.at[cid, 1, pl.ds(sid * RPT, RPT)])


@functools.partial(
    pl.kernel,
    out_type=jax.ShapeDtypeStruct((NC, NPAD, 128), jnp.float32),
    mesh=_MESH,
    scratch_types=[
        pltpu.VMEM((EBATCH,), jnp.int32),
        pltpu.VMEM((EBATCH,), jnp.int32),
        pltpu.VMEM((EBATCH, 128), jnp.float32),
        pltpu.VMEM_SHARED((NPAD, 128), jnp.float32),
        pltpu.SemaphoreType.DMA,
    ],
)
def _prop_kernel(m_hbm, src_hbm, dst_hbm, zrow_hbm, out_hbm,
                 srcb, dstb, rows, acc_sh, sem):
    cid = lax.axis_index("c")
    sid = lax.axis_index("s")
    wid = sid * NC + cid
    pltpu.sync_copy(zrow_hbm, acc_sh.at[pl.ds(sid * RPT, RPT)])
    plsc.subcore_barrier()

    base0 = wid * EPW

    def body(j, carry):
        base = base0 + j * EBATCH
        pltpu.sync_copy(src_hbm.at[pl.ds(base, EBATCH)], srcb)
        pltpu.sync_copy(dst_hbm.at[pl.ds(base, EBATCH)], dstb)
        pltpu.async_copy(m_hbm.at[srcb], rows, sem).wait()
        pltpu.sync_copy(rows, acc_sh.at[dstb], add=True)
        return carry

    lax.fori_loop(0, NBATCH, body, 0)
    plsc.subcore_barrier()
    pltpu.sync_copy(acc_sh.at[pl.ds(sid * RPT, RPT)],
                    out_hbm.at[cid, pl.ds(sid * RPT, RPT)])


BLKR = 1280
GRID = NPAD // BLKR


def _tc1_body(degs_ref, x_ref, w_ref, m0_ref, ns_ref, nd_ref):
    degs = degs_ref[...]
    dego = degs[:, 0:1] + degs[:, 2:3]
    degi = degs[:, 1:2] + degs[:, 3:4]
    ns = jnp.where(dego > 0, lax.rsqrt(jnp.maximum(dego, 1.0)), 0.0)
    nd = jnp.where(degi > 0, lax.rsqrt(jnp.maximum(degi, 1.0)), 0.0)
    ns_ref[...] = ns
    nd_ref[...] = nd
    xw = jnp.dot(x_ref[...], w_ref[...], preferred_element_type=jnp.float32)
    m0_ref[...] = xw * ns


def _tc2_body(agg_ref, nd_ref, ns_ref, b_ref, m1_ref):
    a = agg_ref[0] + agg_ref[1]
    h = jnp.maximum(a * nd_ref[...] + b_ref[...], 0.0)
    m1_ref[...] = h * ns_ref[...]


def _tc3_body(agg_ref, nd_ref, wc_ref, bc_ref, ws_ref, bs_ref, x1_ref, x2_ref):
    g = (agg_ref[0] + agg_ref[1]) * nd_ref[...]
    x1_ref[...] = jnp.maximum(
        jnp.dot(g, wc_ref[...], preferred_element_type=jnp.float32) + bc_ref[...], 0.0)
    x2_ref[...] = jnp.maximum(
        jnp.dot(g, ws_ref[...], preferred_element_type=jnp.float32) + bs_ref[...], 0.0)


def _row_spec(cols):
    return pl.BlockSpec((BLKR, cols), lambda b: (b, 0))


def _full_spec(shape):
    nd = len(shape)
    return pl.BlockSpec(shape, lambda b: (0,) * nd)


def kernel(features, edge_index, W_in, b_in, W_c3, b_c3, W_s3, b_s3):
    src = edge_index[0]
    dst = edge_index[1]
    xpad = jnp.zeros((NPAD, 128), jnp.float32).at[:N_NODES].set(features)
    ones8 = jnp.ones((EBATCH, 8), jnp.float32)
    zero8 = jnp.zeros((RPT, 8), jnp.float32)
    zrow = jnp.zeros((RPT, 128), jnp.float32)

    degs = _degree_kernel(src, dst, ones8, zero8)
    degs_t = degs[:, :, :, 0].reshape(NC * 2, NPAD).T

    m0, ns, nd = pl.pallas_call(
        _tc1_body,
        grid=(GRID,),
        in_specs=[_row_spec(4), _row_spec(128), _full_spec((128, 128))],
        out_specs=[_row_spec(128), _row_spec(1), _row_spec(1)],
        out_shape=[
            jax.ShapeDtypeStruct((NPAD, 128), jnp.float32),
            jax.ShapeDtypeStruct((NPAD, 1), jnp.float32),
            jax.ShapeDtypeStruct((NPAD, 1), jnp.float32),
        ],
    )(degs_t, xpad, W_in)

    agg1 = _prop_kernel(m0, src, dst, zrow)

    m1 = pl.pallas_call(
        _tc2_body,
        grid=(GRID,),
        in_specs=[
            pl.BlockSpec((NC, BLKR, 128), lambda b: (0, b, 0)),
            _row_spec(1), _row_spec(1), _full_spec((1, 128)),
        ],
        out_specs=_row_spec(128),
        out_shape=jax.ShapeDtypeStruct((NPAD, 128), jnp.float32),
    )(agg1, nd, ns, b_in.reshape(1, 128))

    agg2 = _prop_kernel(m1, src, dst, zrow)

    x1, x2 = pl.pallas_call(
        _tc3_body,
        grid=(GRID,),
        in_specs=[
            pl.BlockSpec((NC, BLKR, 128), lambda b: (0, b, 0)),
            _row_spec(1),
            _full_spec((128, 64)), _full_spec((1, 64)),
            _full_spec((128, 64)), _full_spec((1, 64)),
        ],
        out_specs=[_row_spec(64), _row_spec(64)],
        out_shape=[
            jax.ShapeDtypeStruct((NPAD, 64), jnp.float32),
            jax.ShapeDtypeStruct((NPAD, 64), jnp.float32),
        ],
    )(agg2, nd, W_c3, b_c3.reshape(1, 64), W_s3, b_s3.reshape(1, 64))

    return (x1[:N_NODES], x2[:N_NODES])


# speedup vs baseline: 6.2542x; 6.2542x over previous
import functools

import jax
import jax.numpy as jnp
from jax import lax
from jax.experimental import pallas as pl
from jax.experimental.pallas import tpu as pltpu
from jax.experimental.pallas import tpu_sc as plsc

N_NODES = 10000
N_EDGES = 320000
NPAD = 10240
NC, NS = 2, 16
NW = NC * NS
EPW = N_EDGES // NW
EBATCH = 80
NBATCH = EPW // EBATCH
RPT = NPAD // NS


@functools.cache
def _degree_kernel_fn():
    mesh = plsc.VectorSubcoreMesh(core_axis_name="c", subcore_axis_name="s", num_cores=NC, num_subcores=NS)
    return functools.partial(
        pl.kernel,
        out_type=jax.ShapeDtypeStruct((NW, 2, NPAD), jnp.float32),
        mesh=mesh,
        compiler_params=pltpu.CompilerParams(needs_layout_passes=False),
        scratch_types=[
            pltpu.VMEM((EBATCH,), jnp.int32),
            pltpu.VMEM((EBATCH,), jnp.int32),
            pltpu.VMEM((NPAD,), jnp.float32),
            pltpu.VMEM((NPAD,), jnp.float32),
        ],
    )(_degree_body)


def _degree_body(src_hbm, dst_hbm, out_hbm, srcb, dstb, dego_l, degi_l):
    cid = lax.axis_index("c")
    sid = lax.axis_index("s")
    wid = sid * NC + cid
    z16 = jnp.zeros((16,), jnp.float32)

    def zbody(j, carry):
        dego_l[pl.ds(j * 16, 16)] = z16
        degi_l[pl.ds(j * 16, 16)] = z16
        return carry

    lax.fori_loop(0, NPAD // 16, zbody, 0)
    ones16 = jnp.ones((16,), jnp.float32)
    base0 = wid * EPW

    def body(j, carry):
        base = base0 + j * EBATCH
        pltpu.sync_copy(src_hbm.at[pl.ds(base, EBATCH)], srcb)
        pltpu.sync_copy(dst_hbm.at[pl.ds(base, EBATCH)], dstb)
        for k in range(EBATCH // 16):
            sl = pl.ds(k * 16, 16)
            plsc.addupdate_scatter(dego_l, [srcb[sl]], ones16)
            plsc.addupdate_scatter(degi_l, [dstb[sl]], ones16)
        return carry

    lax.fori_loop(0, NBATCH, body, 0)
    pltpu.sync_copy(dego_l, out_hbm.at[wid, 0])
    pltpu.sync_copy(degi_l, out_hbm.at[wid, 1])


@functools.cache
def _prop_kernel_fn():
    mesh = plsc.VectorSubcoreMesh(core_axis_name="c", subcore_axis_name="s", num_cores=NC, num_subcores=NS)
    return functools.partial(
        pl.kernel,
        out_type=jax.ShapeDtypeStruct((NC, NPAD, 128), jnp.float32),
        mesh=mesh,
        scratch_types=[
            pltpu.VMEM((EBATCH,), jnp.int32),
            pltpu.VMEM((EBATCH,), jnp.int32),
            pltpu.VMEM((EBATCH, 128), jnp.float32),
            pltpu.VMEM_SHARED((NPAD, 128), jnp.float32),
            pltpu.SemaphoreType.DMA,
        ],
    )(_prop_body)


def _prop_body(m_hbm, src_hbm, dst_hbm, zrow_hbm, out_hbm,
               srcb, dstb, rows, acc_sh, sem):
    cid = lax.axis_index("c")
    sid = lax.axis_index("s")
    wid = sid * NC + cid
    pltpu.sync_copy(zrow_hbm, acc_sh.at[pl.ds(sid * RPT, RPT)])
    plsc.subcore_barrier()

    base0 = wid * EPW

    def body(j, carry):
        base = base0 + j * EBATCH
        pltpu.sync_copy(src_hbm.at[pl.ds(base, EBATCH)], srcb)
        pltpu.sync_copy(dst_hbm.at[pl.ds(base, EBATCH)], dstb)
        pltpu.async_copy(m_hbm.at[srcb], rows, sem).wait()
        pltpu.sync_copy(rows, acc_sh.at[dstb], add=True)
        return carry

    lax.fori_loop(0, NBATCH, body, 0)
    plsc.subcore_barrier()
    pltpu.sync_copy(acc_sh.at[pl.ds(sid * RPT, RPT)],
                    out_hbm.at[cid, pl.ds(sid * RPT, RPT)])


BLKR = 1280
GRID = NPAD // BLKR


def _tc1_body(dego_ref, degi_ref, x_ref, w_ref, m0_ref, ns_ref, nd_ref):
    dego = jnp.sum(dego_ref[...], axis=1, keepdims=True)
    degi = jnp.sum(degi_ref[...], axis=1, keepdims=True)
    ns = jnp.where(dego > 0, lax.rsqrt(jnp.maximum(dego, 1.0)), 0.0)
    nd = jnp.where(degi > 0, lax.rsqrt(jnp.maximum(degi, 1.0)), 0.0)
    ns_ref[...] = ns
    nd_ref[...] = nd
    xw = jnp.dot(x_ref[...], w_ref[...], preferred_element_type=jnp.float32)
    m0_ref[...] = xw * ns


def _tc2_body(agg_ref, nd_ref, ns_ref, b_ref, m1_ref):
    a = agg_ref[0] + agg_ref[1]
    h = jnp.maximum(a * nd_ref[...] + b_ref[...], 0.0)
    m1_ref[...] = h * ns_ref[...]


def _tc3_body(agg_ref, nd_ref, wc_ref, bc_ref, ws_ref, bs_ref, x1_ref, x2_ref):
    g = (agg_ref[0] + agg_ref[1]) * nd_ref[...]
    x1_ref[...] = jnp.maximum(
        jnp.dot(g, wc_ref[...], preferred_element_type=jnp.float32) + bc_ref[...], 0.0)
    x2_ref[...] = jnp.maximum(
        jnp.dot(g, ws_ref[...], preferred_element_type=jnp.float32) + bs_ref[...], 0.0)


def _row_spec(cols):
    return pl.BlockSpec((BLKR, cols), lambda b: (b, 0))


def _full_spec(shape):
    nd = len(shape)
    return pl.BlockSpec(shape, lambda b: (0,) * nd)


def kernel(features, edge_index, W_in, b_in, W_c3, b_c3, W_s3, b_s3):
    src = edge_index[0]
    dst = edge_index[1]
    xpad = jnp.zeros((NPAD, 128), jnp.float32).at[:N_NODES].set(features)
    zrow = jnp.zeros((RPT, 128), jnp.float32)

    degs = _degree_kernel_fn()(src, dst)
    dego_t = degs[:, 0, :].T
    degi_t = degs[:, 1, :].T

    m0, ns, nd = pl.pallas_call(
        _tc1_body,
        grid=(GRID,),
        in_specs=[_row_spec(NW), _row_spec(NW), _row_spec(128), _full_spec((128, 128))],
        out_specs=[_row_spec(128), _row_spec(1), _row_spec(1)],
        out_shape=[
            jax.ShapeDtypeStruct((NPAD, 128), jnp.float32),
            jax.ShapeDtypeStruct((NPAD, 1), jnp.float32),
            jax.ShapeDtypeStruct((NPAD, 1), jnp.float32),
        ],
    )(dego_t, degi_t, xpad, W_in)

    agg1 = _prop_kernel_fn()(m0, src, dst, zrow)

    m1 = pl.pallas_call(
        _tc2_body,
        grid=(GRID,),
        in_specs=[
            pl.BlockSpec((NC, BLKR, 128), lambda b: (0, b, 0)),
            _row_spec(1), _row_spec(1), _full_spec((1, 128)),
        ],
        out_specs=_row_spec(128),
        out_shape=jax.ShapeDtypeStruct((NPAD, 128), jnp.float32),
    )(agg1, nd, ns, b_in.reshape(1, 128))

    agg2 = _prop_kernel_fn()(m1, src, dst, zrow)

    x1, x2 = pl.pallas_call(
        _tc3_body,
        grid=(GRID,),
        in_specs=[
            pl.BlockSpec((NC, BLKR, 128), lambda b: (0, b, 0)),
            _row_spec(1),
            _full_spec((128, 64)), _full_spec((1, 64)),
            _full_spec((128, 64)), _full_spec((1, 64)),
        ],
        out_specs=[_row_spec(64), _row_spec(64)],
        out_shape=[
            jax.ShapeDtypeStruct((NPAD, 64), jnp.float32),
            jax.ShapeDtypeStruct((NPAD, 64), jnp.float32),
        ],
    )(agg2, nd, W_c3, b_c3.reshape(1, 64), W_s3, b_s3.reshape(1, 64))

    return (x1[:N_NODES], x2[:N_NODES])


# speedup vs baseline: 11.7247x; 1.8747x over previous
import functools

import jax
import jax.numpy as jnp
from jax import lax
from jax.experimental import pallas as pl
from jax.experimental.pallas import tpu as pltpu
from jax.experimental.pallas import tpu_sc as plsc

N_NODES = 10000
N_EDGES = 320000
NPAD = 10240
NC, NS = 2, 16
NW = NC * NS
EPW = N_EDGES // NW
EBATCH = 80
NBATCH = EPW // EBATCH
NCHUNK = 5
CHB = NBATCH // NCHUNK
RPT = NPAD // NS


@functools.cache
def _degree_kernel_fn():
    mesh = plsc.VectorSubcoreMesh(core_axis_name="c", subcore_axis_name="s", num_cores=NC, num_subcores=NS)
    return functools.partial(
        pl.kernel,
        out_type=jax.ShapeDtypeStruct((NW, 2, NPAD), jnp.float32),
        mesh=mesh,
        compiler_params=pltpu.CompilerParams(needs_layout_passes=False),
        scratch_types=[
            pltpu.VMEM((EPW,), jnp.int32),
            pltpu.VMEM((EPW,), jnp.int32),
            pltpu.VMEM((NPAD,), jnp.float32),
            pltpu.VMEM((NPAD,), jnp.float32),
        ],
    )(_degree_body)


def _degree_body(src_hbm, dst_hbm, zflat_hbm, out_hbm, srcb, dstb, dego_l, degi_l):
    cid = lax.axis_index("c")
    sid = lax.axis_index("s")
    wid = sid * NC + cid
    pltpu.sync_copy(zflat_hbm, dego_l)
    pltpu.sync_copy(zflat_hbm, degi_l)
    pltpu.sync_copy(src_hbm.at[pl.ds(wid * EPW, EPW)], srcb)
    pltpu.sync_copy(dst_hbm.at[pl.ds(wid * EPW, EPW)], dstb)
    ones16 = jnp.ones((16,), jnp.float32)

    def body(j, carry):
        sl = pl.ds(j * 16, 16)
        plsc.addupdate_scatter(dego_l, [srcb[sl]], ones16)
        plsc.addupdate_scatter(degi_l, [dstb[sl]], ones16)
        return carry

    lax.fori_loop(0, EPW // 16, body, 0)
    pltpu.sync_copy(dego_l, out_hbm.at[wid, 0])
    pltpu.sync_copy(degi_l, out_hbm.at[wid, 1])


@functools.cache
def _prop_kernel_fn():
    mesh = plsc.VectorSubcoreMesh(core_axis_name="c", subcore_axis_name="s", num_cores=NC, num_subcores=NS)
    return functools.partial(
        pl.kernel,
        out_type=jax.ShapeDtypeStruct((NC, NPAD, 128), jnp.float32),
        mesh=mesh,
        scratch_types=[
            pltpu.VMEM((CHB, EBATCH), jnp.int32),
            pltpu.VMEM((CHB, EBATCH), jnp.int32),
            pltpu.VMEM((EBATCH, 128), jnp.float32),
            pltpu.VMEM((EBATCH, 128), jnp.float32),
            pltpu.VMEM_SHARED((NPAD, 128), jnp.float32),
            pltpu.SemaphoreType.DMA,
            pltpu.SemaphoreType.DMA,
        ],
    )(_prop_body)


def _prop_body(m_hbm, src4_hbm, dst4_hbm, zrow_hbm, out_hbm,
               src2, dst2, rows0, rows1, acc_sh, sem0, sem1):
    cid = lax.axis_index("c")
    sid = lax.axis_index("s")
    wid = sid * NC + cid
    pltpu.sync_copy(zrow_hbm, acc_sh.at[pl.ds(sid * RPT, RPT)])
    plsc.subcore_barrier()

    def fire(jj, rows, sem):
        pltpu.async_copy(m_hbm.at[src2.at[jj]], rows, sem)

    def wait(jj, rows, sem):
        pltpu.make_async_copy(m_hbm.at[src2.at[jj]], rows, sem).wait()

    def scat(jj, rows):
        pltpu.sync_copy(rows, acc_sh.at[dst2.at[jj]], add=True)

    def chunk_body(c, carry):
        pltpu.sync_copy(src4_hbm.at[wid, c], src2)
        pltpu.sync_copy(dst4_hbm.at[wid, c], dst2)
        fire(0, rows0, sem0)

        def body(i, carry2):
            j = 2 * i
            wait(j, rows0, sem0)
            fire(j + 1, rows1, sem1)
            scat(j, rows0)
            wait(j + 1, rows1, sem1)
            fire(j + 2, rows0, sem0)
            scat(j + 1, rows1)
            return carry2

        lax.fori_loop(0, (CHB - 1) // 2, body, 0)
        wait(CHB - 1, rows0, sem0)
        scat(CHB - 1, rows0)
        return carry

    lax.fori_loop(0, NCHUNK, chunk_body, 0)
    plsc.subcore_barrier()
    pltpu.sync_copy(acc_sh.at[pl.ds(sid * RPT, RPT)],
                    out_hbm.at[cid, pl.ds(sid * RPT, RPT)])


BLKR = 1280
GRID = NPAD // BLKR


def _tc1_body(dego_ref, degi_ref, x_ref, w_ref, m0_ref, ns_ref, nd_ref):
    dego = jnp.sum(dego_ref[...], axis=1, keepdims=True)
    degi = jnp.sum(degi_ref[...], axis=1, keepdims=True)
    ns = jnp.where(dego > 0, lax.rsqrt(jnp.maximum(dego, 1.0)), 0.0)
    nd = jnp.where(degi > 0, lax.rsqrt(jnp.maximum(degi, 1.0)), 0.0)
    ns_ref[...] = ns
    nd_ref[...] = nd
    xw = jnp.dot(x_ref[...], w_ref[...], preferred_element_type=jnp.float32)
    m0_ref[...] = xw * ns


def _tc2_body(agg_ref, nd_ref, ns_ref, b_ref, m1_ref):
    a = agg_ref[0] + agg_ref[1]
    h = jnp.maximum(a * nd_ref[...] + b_ref[...], 0.0)
    m1_ref[...] = h * ns_ref[...]


def _tc3_body(agg_ref, nd_ref, wc_ref, bc_ref, ws_ref, bs_ref, x1_ref, x2_ref):
    g = (agg_ref[0] + agg_ref[1]) * nd_ref[...]
    x1_ref[...] = jnp.maximum(
        jnp.dot(g, wc_ref[...], preferred_element_type=jnp.float32) + bc_ref[...], 0.0)
    x2_ref[...] = jnp.maximum(
        jnp.dot(g, ws_ref[...], preferred_element_type=jnp.float32) + bs_ref[...], 0.0)


def _row_spec(cols):
    return pl.BlockSpec((BLKR, cols), lambda b: (b, 0))


def _full_spec(shape):
    nd = len(shape)
    return pl.BlockSpec(shape, lambda b: (0,) * nd)


def kernel(features, edge_index, W_in, b_in, W_c3, b_c3, W_s3, b_s3):
    src = edge_index[0]
    dst = edge_index[1]
    xpad = jnp.zeros((NPAD, 128), jnp.float32).at[:N_NODES].set(features)
    zrow = jnp.zeros((RPT, 128), jnp.float32)
    zflat = jnp.zeros((NPAD,), jnp.float32)
    src4 = src.reshape(NW, NCHUNK, CHB, EBATCH)
    dst4 = dst.reshape(NW, NCHUNK, CHB, EBATCH)

    degs = _degree_kernel_fn()(src, dst, zflat)
    dego_t = degs[:, 0, :].T
    degi_t = degs[:, 1, :].T

    m0, ns, nd = pl.pallas_call(
        _tc1_body,
        grid=(GRID,),
        in_specs=[_row_spec(NW), _row_spec(NW), _row_spec(128), _full_spec((128, 128))],
        out_specs=[_row_spec(128), _row_spec(1), _row_spec(1)],
        out_shape=[
            jax.ShapeDtypeStruct((NPAD, 128), jnp.float32),
            jax.ShapeDtypeStruct((NPAD, 1), jnp.float32),
            jax.ShapeDtypeStruct((NPAD, 1), jnp.float32),
        ],
    )(dego_t, degi_t, xpad, W_in)

    agg1 = _prop_kernel_fn()(m0, src4, dst4, zrow)

    m1 = pl.pallas_call(
        _tc2_body,
        grid=(GRID,),
        in_specs=[
            pl.BlockSpec((NC, BLKR, 128), lambda b: (0, b, 0)),
            _row_spec(1), _row_spec(1), _full_spec((1, 128)),
        ],
        out_specs=_row_spec(128),
        out_shape=jax.ShapeDtypeStruct((NPAD, 128), jnp.float32),
    )(agg1, nd, ns, b_in.reshape(1, 128))

    agg2 = _prop_kernel_fn()(m1, src4, dst4, zrow)

    x1, x2 = pl.pallas_call(
        _tc3_body,
        grid=(GRID,),
        in_specs=[
            pl.BlockSpec((NC, BLKR, 128), lambda b: (0, b, 0)),
            _row_spec(1),
            _full_spec((128, 64)), _full_spec((1, 64)),
            _full_spec((128, 64)), _full_spec((1, 64)),
        ],
        out_specs=[_row_spec(64), _row_spec(64)],
        out_shape=[
            jax.ShapeDtypeStruct((NPAD, 64), jnp.float32),
            jax.ShapeDtypeStruct((NPAD, 64), jnp.float32),
        ],
    )(agg2, nd, W_c3, b_c3.reshape(1, 64), W_s3, b_s3.reshape(1, 64))

    return (x1[:N_NODES], x2[:N_NODES])
